# Initial kernel scaffold; baseline (speedup 1.0000x reference)
import jax
import jax.numpy as jnp
from jax.experimental import pallas as pl


def kernel(inputs, tables, W):
    raise NotImplementedError("write your pallas kernel here")


# speedup vs baseline: 1.4270x; 1.4270x over previous
import functools

import numpy as np
import jax
import jax.numpy as jnp
from jax import lax
from jax.experimental import pallas as pl
from jax.experimental.pallas import tpu as pltpu
from jax.experimental.pallas import tpu_sc as plsc

N_LEVELS = 12
N_FEATURES = 4
TABLE_SIZE = 524288
BASE_RES = 16
MAX_RES = 2048
FEATURE_DIM = N_LEVELS * N_FEATURES
OUT_DIM = FEATURE_DIM + 1
N_POINTS = 262144

P2 = int(np.uint32(2654435761).view(np.int32))
P3 = 805459861

NC = 2
NS = 16
L = 16


def _resolutions(n_levels=N_LEVELS):
    b = np.exp((np.log(MAX_RES) - np.log(BASE_RES)) / (N_LEVELS - 1))
    return [int(np.floor(BASE_RES * (b ** l))) for l in range(n_levels)]


def _build_encode(n_points, n_levels, table_size, chunk):
    nw = NC * NS
    pts_per_w = n_points // nw
    assert pts_per_w % chunk == 0
    n_chunks = pts_per_w // chunk
    groups = chunk // L
    rows_n = 8 * chunk
    mask = table_size - 1
    res_list = _resolutions(n_levels)
    feat_dim = n_levels * N_FEATURES

    mesh = plsc.VectorSubcoreMesh(core_axis_name="c", subcore_axis_name="s",
                                  num_cores=NC, num_subcores=NS)

    @functools.partial(
        pl.kernel,
        out_type=jax.ShapeDtypeStruct((feat_dim, n_points), jnp.float32),
        mesh=mesh,
        scratch_types=(
            [pltpu.VMEM((3, chunk), jnp.float32)]
            + [pltpu.VMEM((rows_n,), jnp.int32)
               for _ in range(n_levels)]
            + [pltpu.VMEM((n_levels, rows_n), jnp.float32)]
            + [pltpu.VMEM((rows_n,), jnp.float32)
               for _ in range(2 * N_FEATURES)]
            + [pltpu.VMEM((feat_dim, chunk), jnp.float32)]
            + [pltpu.SemaphoreType.DMA((2,))]
        ),
    )
    def encode(xyz_hbm, t0_hbm, t1_hbm, t2_hbm, t3_hbm, feats_hbm,
               *refs):
        xyz_v = refs[0]
        idx_refs = refs[1:1 + n_levels]
        w_v = refs[1 + n_levels]
        row_refs = refs[2 + n_levels:2 + n_levels + 2 * N_FEATURES]
        feats_v = refs[2 + n_levels + 2 * N_FEATURES]
        sems = refs[3 + n_levels + 2 * N_FEATURES]

        wid = lax.axis_index("s") * NC + lax.axis_index("c")
        iota = lax.iota(jnp.int32, L)
        planes = (t0_hbm, t1_hbm, t2_hbm, t3_hbm)

        def fire(l, ph):
            for j in range(N_FEATURES):
                pltpu.make_async_copy(
                    planes[j].at[idx_refs[l]], row_refs[ph * N_FEATURES + j],
                    sems.at[ph]).start()

        def wait(l, ph):
            for j in range(N_FEATURES):
                pltpu.make_async_copy(
                    planes[j].at[idx_refs[l]], row_refs[ph * N_FEATURES + j],
                    sems.at[ph]).wait()

        def chunk_body(ci, _):
            base = wid * pts_per_w + ci * chunk
            pltpu.sync_copy(xyz_hbm.at[:, pl.ds(base, chunk)], xyz_v)

            def group_a(g, _):
                off = g * L
                x = xyz_v[0, pl.ds(off, L)]
                y = xyz_v[1, pl.ds(off, L)]
                z = xyz_v[2, pl.ds(off, L)]
                for l in range(n_levels):
                    res = float(res_list[l])
                    sx = x * res
                    sy = y * res
                    sz = z * res
                    ix = sx.astype(jnp.int32)
                    iy = sy.astype(jnp.int32)
                    iz = sz.astype(jnp.int32)
                    fx = sx - ix.astype(jnp.float32)
                    fy = sy - iy.astype(jnp.float32)
                    fz = sz - iz.astype(jnp.float32)
                    gx = 1.0 - fx
                    gy = 1.0 - fy
                    gz = 1.0 - fz
                    hy0 = iy * P2
                    hz0 = iz * P3
                    hx1 = ix + 1
                    hy1 = hy0 + P2
                    hz1 = hz0 + P3
                    lbase = l * table_size
                    for corner in range(8):
                        hx = hx1 if (corner & 1) else ix
                        hy = hy1 if (corner & 2) else hy0
                        hz = hz1 if (corner & 4) else hz0
                        h = ((hx ^ hy ^ hz) & mask) + lbase
                        w = ((fx if (corner & 1) else gx)
                             * (fy if (corner & 2) else gy)
                             * (fz if (corner & 4) else gz))
                        pos = corner * chunk + off
                        idx_refs[l][pl.ds(pos, L)] = h
                        w_v[l, pl.ds(pos, L)] = w
                return 0

            lax.fori_loop(0, groups, group_a, 0, unroll=False)

            fire(0, 0)
            for l in range(n_levels):
                ph = l % 2
                if l + 1 < n_levels:
                    fire(l + 1, 1 - ph)
                wait(l, ph)

                def group_c(g, _, l=l, ph=ph):
                    off = g * L
                    acc = [jnp.zeros((L,), jnp.float32)
                           for _ in range(N_FEATURES)]
                    for corner in range(8):
                        pos = corner * chunk + off
                        wv = w_v[l, pl.ds(pos, L)]
                        for j in range(N_FEATURES):
                            fj = row_refs[ph * N_FEATURES + j][pl.ds(pos, L)]
                            acc[j] = acc[j] + wv * fj
                    for j in range(N_FEATURES):
                        feats_v[N_FEATURES * l + j, pl.ds(off, L)] = acc[j]
                    return 0

                lax.fori_loop(0, groups, group_c, 0, unroll=False)

            pltpu.sync_copy(feats_v, feats_hbm.at[:, pl.ds(base, chunk)])
            return 0

        lax.fori_loop(0, n_chunks, chunk_body, 0, unroll=False)

    return encode


def _matmul(feats_t, w, block_n=2048):
    n = feats_t.shape[1]
    fd, od = w.shape

    def mm_kernel(f_ref, w_ref, o_ref):
        o_ref[...] = lax.dot_general(
            f_ref[...], w_ref[...], (((0,), (0,)), ((), ())),
            preferred_element_type=jnp.float32)

    return pl.pallas_call(
        mm_kernel,
        grid=(n // block_n,),
        in_specs=[
            pl.BlockSpec((fd, block_n), lambda i: (0, i)),
            pl.BlockSpec((fd, od), lambda i: (0, 0)),
        ],
        out_specs=pl.BlockSpec((block_n, od), lambda i: (i, 0)),
        out_shape=jax.ShapeDtypeStruct((n, od), jnp.float32),
    )(feats_t, w)


_encode_full = _build_encode(N_POINTS, N_LEVELS, TABLE_SIZE, chunk=256)


def kernel(inputs, tables, W):
    xyz = inputs.T
    planes = jnp.transpose(tables, (2, 0, 1)).reshape(N_FEATURES, -1)
    feats_t = _encode_full(xyz, planes[0], planes[1], planes[2], planes[3])
    return _matmul(feats_t, W)
